# baseline (speedup 1.0000x reference)
import functools

import jax
import jax.numpy as jnp
from jax import lax
from jax.experimental import pallas as pl
from jax.experimental.pallas import tpu as pltpu
from jax.experimental.pallas import tpu_sc as plsc

N = 10000
E = 320000
D = 128

NC = 2
NS = 16
NW = NC * NS
B = 128
K = 160
HD = D // 2
EP = NW * K * B
NP = 10240
R = 1024
GRID = NP // R
RPT = 2 * NP // NS
NBUF = 2
SB = 40


def _sc_aggregate(table, src_g, dst_g, zeros):
    mesh = plsc.VectorSubcoreMesh(
        core_axis_name="c", subcore_axis_name="s", num_cores=NC, num_subcores=NS
    )

    @functools.partial(
        pl.kernel,
        out_type=jax.ShapeDtypeStruct((NC, 2 * NP, HD), jnp.float32),
        mesh=mesh,
        compiler_params=pltpu.CompilerParams(use_tc_tiling_on_sc=False),
        scratch_types=[
            pltpu.VMEM((SB, B), jnp.int32),
            pltpu.VMEM((SB, B), jnp.int32),
            pltpu.VMEM((B, HD), jnp.float32),
            pltpu.VMEM((B, HD), jnp.float32),
            pltpu.SemaphoreType.DMA,
            pltpu.SemaphoreType.DMA,
            pltpu.VMEM_SHARED((2 * NP, HD), jnp.float32),
        ],
    )
    def k(table_hbm, src_hbm, dst_hbm, zeros_hbm, out_hbm, src_c, dst_c,
          rows0, rows1, gsem0, gsem1, acc):
        rows = (rows0, rows1)
        gsem = (gsem0, gsem1)
        c = lax.axis_index("c")
        s = lax.axis_index("s")
        wid = s * NC + c
        pltpu.sync_copy(zeros_hbm.at[pl.ds(s * RPT, RPT)], acc.at[pl.ds(s * RPT, RPT)])
        plsc.subcore_barrier()

        def super_(t, carry):
            pltpu.sync_copy(src_hbm.at[wid, pl.ds(t * SB, SB)], src_c)
            pltpu.sync_copy(dst_hbm.at[wid, pl.ds(t * SB, SB)], dst_c)
            for b in range(NBUF):
                pltpu.async_copy(table_hbm.at[src_c.at[b]], rows[b], gsem[b])

            def pair_(p, carry2):
                for b in range(NBUF):
                    i = p * NBUF + b
                    pltpu.make_async_copy(table_hbm.at[src_c.at[i]], rows[b], gsem[b]).wait()
                    pltpu.sync_copy(rows[b], acc.at[dst_c.at[i]], add=True)
                    ni = i + NBUF

                    @pl.when(ni < SB)
                    def _():
                        pltpu.async_copy(table_hbm.at[src_c.at[ni]], rows[b], gsem[b])
                return carry2

            lax.fori_loop(0, SB // NBUF, pair_, 0)
            return carry

        lax.fori_loop(0, K // SB, super_, 0)
        plsc.subcore_barrier()
        pltpu.sync_copy(acc.at[pl.ds(s * RPT, RPT)], out_hbm.at[c, pl.ds(s * RPT, RPT)])

    return k(table, src_g, dst_g, zeros)


def _mlp_body(x_ref, a0_ref, a1_ref, w1_ref, b1_ref, w2_ref, b2_ref, o_ref, *, final_relu):
    z = x_ref[...] + a0_ref[...] + a1_ref[...]
    z = jnp.dot(z, w1_ref[...], preferred_element_type=jnp.float32) + b1_ref[...]
    z = jnp.maximum(z, 0.0)
    z = jnp.dot(z, w2_ref[...], preferred_element_type=jnp.float32) + b2_ref[...]
    if final_relu:
        z = jnp.maximum(z, 0.0)
    rows = pl.program_id(0) * R + lax.broadcasted_iota(jnp.int32, (R, D), 0)
    o_ref[...] = jnp.where(rows < N, z, 0.0)


def _tc_mlp(h, acc, W1, b1, W2, b2, final_relu):
    row_spec = pl.BlockSpec((R, D), lambda i: (i, 0))
    full_spec = pl.BlockSpec((D, D), lambda i: (0, 0))
    bias_spec = pl.BlockSpec((1, D), lambda i: (0, 0))
    return pl.pallas_call(
        functools.partial(_mlp_body, final_relu=final_relu),
        grid=(GRID,),
        in_specs=[row_spec, row_spec, row_spec, full_spec, bias_spec, full_spec, bias_spec],
        out_specs=row_spec,
        out_shape=jax.ShapeDtypeStruct((NP, D), jnp.float32),
    )(h, acc[0], acc[1], W1, b1.reshape(1, D), W2, b2.reshape(1, D))


def kernel(x, edge_index, W1_0, b1_0, W2_0, b2_0, W1_1, b1_1, W2_1, b2_1,
           W1_2, b1_2, W2_2, b2_2):
    src = edge_index[0]
    dst = edge_index[1]
    half = jnp.arange(2, dtype=jnp.int32)
    src2 = (2 * src[:, None] + half).reshape(-1)
    dst2 = (2 * dst[:, None] + half).reshape(-1)
    src_p = jnp.full((EP,), 2 * N, dtype=jnp.int32).at[: 2 * E].set(src2)
    dst_p = jnp.zeros((EP,), dtype=jnp.int32).at[: 2 * E].set(dst2)
    src_g = src_p.reshape(NW, K, B)
    dst_g = dst_p.reshape(NW, K, B)
    zeros = jnp.zeros((2 * NP, HD), dtype=jnp.float32)

    h = jnp.zeros((NP, D), dtype=jnp.float32).at[:N].set(x)
    weights = [(W1_0, b1_0, W2_0, b2_0), (W1_1, b1_1, W2_1, b2_1), (W1_2, b1_2, W2_2, b2_2)]
    for l, (W1, b1, W2, b2) in enumerate(weights):
        acc = _sc_aggregate(h.reshape(2 * NP, HD), src_g, dst_g, zeros)
        h = _tc_mlp(h, acc.reshape(NC, NP, D), W1, b1, W2, b2, final_relu=(l < 2))
    return h[:N]


# speedup vs baseline: 1.7924x; 1.7924x over previous
import functools

import jax
import jax.numpy as jnp
from jax import lax
from jax.experimental import pallas as pl
from jax.experimental.pallas import tpu as pltpu
from jax.experimental.pallas import tpu_sc as plsc

N = 10000
E = 320000
D = 128

NC = 2
NS = 16
NW = NC * NS
B = 128
K = 80
EP = NW * K * B
NP = 10240
R = 1024
GRID = NP // R
RPT = NP // NS
NBUF = 2
SB = 40


def _sc_aggregate(table, src_g, dst_g, zeros):
    mesh = plsc.VectorSubcoreMesh(
        core_axis_name="c", subcore_axis_name="s", num_cores=NC, num_subcores=NS
    )

    @functools.partial(
        pl.kernel,
        out_type=jax.ShapeDtypeStruct((NC, NP, D), jnp.float32),
        mesh=mesh,
        compiler_params=pltpu.CompilerParams(use_tc_tiling_on_sc=False),
        scratch_types=[
            pltpu.VMEM((SB, B), jnp.int32),
            pltpu.VMEM((SB, B), jnp.int32),
            pltpu.VMEM((B, D // 2), jnp.float32),
            pltpu.VMEM((B, D // 2), jnp.float32),
            pltpu.VMEM((B, D), jnp.float32),
            pltpu.SemaphoreType.DMA,
            pltpu.SemaphoreType.DMA,
            pltpu.VMEM_SHARED((NP, D), jnp.float32),
        ],
    )
    def k(table_hbm, src_hbm, dst_hbm, zeros_hbm, out_hbm, src_c, dst_c,
          rows0, rows1, fbuf, gsem0, gsem1, acc):
        rows = (rows0, rows1)
        gsem = (gsem0, gsem1)
        c = lax.axis_index("c")
        s = lax.axis_index("s")
        wid = s * NC + c
        pltpu.sync_copy(zeros_hbm.at[pl.ds(s * RPT, RPT)], acc.at[pl.ds(s * RPT, RPT)])
        plsc.subcore_barrier()

        def super_(t, carry):
            pltpu.sync_copy(src_hbm.at[wid, pl.ds(t * SB, SB)], src_c)
            pltpu.sync_copy(dst_hbm.at[wid, pl.ds(t * SB, SB)], dst_c)
            for b in range(NBUF):
                pltpu.async_copy(table_hbm.at[src_c.at[b]], rows[b], gsem[b])

            def pair_(p, carry2):
                for b in range(NBUF):
                    i = p * NBUF + b
                    pltpu.make_async_copy(table_hbm.at[src_c.at[i]], rows[b], gsem[b]).wait()
                    pltpu.sync_copy(fbuf, acc.at[dst_c.at[i]], add=True)
                    ni = i + NBUF

                    @pl.when(ni < SB)
                    def _():
                        pltpu.async_copy(table_hbm.at[src_c.at[ni]], rows[b], gsem[b])
                return carry2

            lax.fori_loop(0, SB // NBUF, pair_, 0)
            return carry

        lax.fori_loop(0, K // SB, super_, 0)
        plsc.subcore_barrier()
        pltpu.sync_copy(acc.at[pl.ds(s * RPT, RPT)], out_hbm.at[c, pl.ds(s * RPT, RPT)])

    return k(jnp.asarray(table[:, ::2]), src_g, dst_g, zeros)


def _mlp_body(x_ref, a0_ref, a1_ref, w1_ref, b1_ref, w2_ref, b2_ref, o_ref, *, final_relu):
    z = x_ref[...] + a0_ref[...] + a1_ref[...]
    z = jnp.dot(z, w1_ref[...], preferred_element_type=jnp.float32) + b1_ref[...]
    z = jnp.maximum(z, 0.0)
    z = jnp.dot(z, w2_ref[...], preferred_element_type=jnp.float32) + b2_ref[...]
    if final_relu:
        z = jnp.maximum(z, 0.0)
    rows = pl.program_id(0) * R + lax.broadcasted_iota(jnp.int32, (R, D), 0)
    o_ref[...] = jnp.where(rows < N, z, 0.0)


def _tc_mlp(h, acc, W1, b1, W2, b2, final_relu):
    row_spec = pl.BlockSpec((R, D), lambda i: (i, 0))
    full_spec = pl.BlockSpec((D, D), lambda i: (0, 0))
    bias_spec = pl.BlockSpec((1, D), lambda i: (0, 0))
    return pl.pallas_call(
        functools.partial(_mlp_body, final_relu=final_relu),
        grid=(GRID,),
        in_specs=[row_spec, row_spec, row_spec, full_spec, bias_spec, full_spec, bias_spec],
        out_specs=row_spec,
        out_shape=jax.ShapeDtypeStruct((NP, D), jnp.float32),
    )(h, acc[0], acc[1], W1, b1.reshape(1, D), W2, b2.reshape(1, D))


def kernel(x, edge_index, W1_0, b1_0, W2_0, b2_0, W1_1, b1_1, W2_1, b2_1,
           W1_2, b1_2, W2_2, b2_2):
    src = edge_index[0]
    dst = edge_index[1]
    src_p = jnp.full((EP,), N, dtype=jnp.int32).at[:E].set(src)
    dst_p = jnp.zeros((EP,), dtype=jnp.int32).at[:E].set(dst)
    src_g = src_p.reshape(NW, K, B)
    dst_g = dst_p.reshape(NW, K, B)
    zeros = jnp.zeros((NP, D), dtype=jnp.float32)

    h = jnp.zeros((NP, D), dtype=jnp.float32).at[:N].set(x)
    weights = [(W1_0, b1_0, W2_0, b2_0), (W1_1, b1_1, W2_1, b2_1), (W1_2, b1_2, W2_2, b2_2)]
    for l, (W1, b1, W2, b2) in enumerate(weights):
        acc = _sc_aggregate(h, src_g, dst_g, zeros)
        h = _tc_mlp(h, acc, W1, b1, W2, b2, final_relu=(l < 2))
    return h[:N]


# speedup vs baseline: 1.9442x; 1.0847x over previous
import functools

import numpy as np
import jax
import jax.numpy as jnp
from jax import lax
from jax.experimental import pallas as pl
from jax.experimental.pallas import tpu as pltpu
from jax.experimental.pallas import tpu_sc as plsc

N = 10000
E = 320000
D = 128
HW = D // 2

NC = 2
NS = 16
NW = NC * NS
B = 128
K = 80
EP = NW * K * B
NP = 10240
R = 1024
GRID = NP // R
RPT = NP // NS
NBUF = 2
SB = 40

_PERM = np.empty((D,), dtype=np.int32)
for _j in range(D // 32):
    for _q in range(16):
        _PERM[32 * _j + _q] = 32 * _j + 2 * _q
        _PERM[32 * _j + 16 + _q] = 32 * _j + 2 * _q + 1
_PERM.setflags(write=False)


def _sc_aggregate(table, src_g, dst_g, zeros):
    mesh = plsc.VectorSubcoreMesh(
        core_axis_name="c", subcore_axis_name="s", num_cores=NC, num_subcores=NS
    )

    @functools.partial(
        pl.kernel,
        out_type=jax.ShapeDtypeStruct((NC, NP, D), jnp.float32),
        mesh=mesh,
        compiler_params=pltpu.CompilerParams(use_tc_tiling_on_sc=False),
        scratch_types=[
            pltpu.VMEM((SB, B), jnp.int32),
            pltpu.VMEM((SB, B), jnp.int32),
            pltpu.VMEM((B, HW), jnp.int32),
            pltpu.VMEM((B, HW), jnp.int32),
            pltpu.VMEM((B, D), jnp.float32),
            pltpu.SemaphoreType.DMA,
            pltpu.SemaphoreType.DMA,
            pltpu.VMEM_SHARED((NP, D), jnp.float32),
        ],
    )
    def k(table_hbm, src_hbm, dst_hbm, zeros_hbm, out_hbm, src_c, dst_c,
          rows0, rows1, fbuf, gsem0, gsem1, acc):
        rows = (rows0, rows1)
        gsem = (gsem0, gsem1)
        c = lax.axis_index("c")
        s = lax.axis_index("s")
        wid = s * NC + c
        pltpu.sync_copy(zeros_hbm.at[pl.ds(s * RPT, RPT)], acc.at[pl.ds(s * RPT, RPT)])
        plsc.subcore_barrier()

        def widen(buf):
            def row_(r, carry):
                for j in range(D // 32):
                    v = buf[r, pl.ds(16 * j, 16)]
                    sixteen = jnp.full((16,), 16, dtype=jnp.int32)
                    himask = jnp.full((16,), -65536, dtype=jnp.int32)
                    lo = lax.bitcast_convert_type(lax.shift_left(v, sixteen), jnp.float32)
                    hi = lax.bitcast_convert_type(lax.bitwise_and(v, himask), jnp.float32)
                    fbuf[r, pl.ds(32 * j, 16)] = lo
                    fbuf[r, pl.ds(32 * j + 16, 16)] = hi
                return carry

            lax.fori_loop(0, B, row_, 0)

        def super_(t, carry):
            pltpu.sync_copy(src_hbm.at[wid, pl.ds(t * SB, SB)], src_c)
            pltpu.sync_copy(dst_hbm.at[wid, pl.ds(t * SB, SB)], dst_c)
            for b in range(NBUF):
                pltpu.async_copy(table_hbm.at[src_c.at[b]], rows[b], gsem[b])

            def pair_(p, carry2):
                for b in range(NBUF):
                    i = p * NBUF + b
                    pltpu.make_async_copy(table_hbm.at[src_c.at[i]], rows[b], gsem[b]).wait()
                    widen(rows[b])
                    ni = i + NBUF

                    @pl.when(ni < SB)
                    def _():
                        pltpu.async_copy(table_hbm.at[src_c.at[ni]], rows[b], gsem[b])

                    pltpu.sync_copy(fbuf, acc.at[dst_c.at[i]], add=True)
                return carry2

            lax.fori_loop(0, SB // NBUF, pair_, 0)
            return carry

        lax.fori_loop(0, K // SB, super_, 0)
        plsc.subcore_barrier()
        pltpu.sync_copy(acc.at[pl.ds(s * RPT, RPT)], out_hbm.at[c, pl.ds(s * RPT, RPT)])

    return k(table, src_g, dst_g, zeros)


def _mlp_mid_body(x_ref, a0_ref, a1_ref, w1_ref, b1_ref, w2_ref, b2_ref,
                  w2p_ref, b2p_ref, h_ref, xp_ref):
    z = x_ref[...] + a0_ref[...] + a1_ref[...]
    y = jnp.dot(z, w1_ref[...], preferred_element_type=jnp.float32) + b1_ref[...]
    y = jnp.maximum(y, 0.0)
    rows = pl.program_id(0) * R + lax.broadcasted_iota(jnp.int32, (R, D), 0)
    mask = rows < N
    h = jnp.dot(y, w2_ref[...], preferred_element_type=jnp.float32) + b2_ref[...]
    h = jnp.where(mask, jnp.maximum(h, 0.0), 0.0)
    h_ref[...] = h.astype(jnp.bfloat16)
    hp = jnp.dot(y, w2p_ref[...], preferred_element_type=jnp.float32) + b2p_ref[...]
    xp_ref[...] = jnp.where(mask, jnp.maximum(hp, 0.0), 0.0)


def _mlp_last_body(x_ref, a0_ref, a1_ref, w1_ref, b1_ref, w2_ref, b2_ref, o_ref):
    z = x_ref[...] + a0_ref[...] + a1_ref[...]
    y = jnp.dot(z, w1_ref[...], preferred_element_type=jnp.float32) + b1_ref[...]
    y = jnp.maximum(y, 0.0)
    h = jnp.dot(y, w2_ref[...], preferred_element_type=jnp.float32) + b2_ref[...]
    rows = pl.program_id(0) * R + lax.broadcasted_iota(jnp.int32, (R, D), 0)
    o_ref[...] = jnp.where(rows < N, h, 0.0)


def _tc_mlp_mid(xp, acc, W1p, b1, W2, b2, W2p, b2p):
    row_spec = pl.BlockSpec((R, D), lambda i: (i, 0))
    full_spec = pl.BlockSpec((D, D), lambda i: (0, 0))
    bias_spec = pl.BlockSpec((1, D), lambda i: (0, 0))
    return pl.pallas_call(
        _mlp_mid_body,
        grid=(GRID,),
        in_specs=[row_spec, row_spec, row_spec, full_spec, bias_spec,
                  full_spec, bias_spec, full_spec, bias_spec],
        out_specs=[row_spec, row_spec],
        out_shape=[
            jax.ShapeDtypeStruct((NP, D), jnp.bfloat16),
            jax.ShapeDtypeStruct((NP, D), jnp.float32),
        ],
    )(xp, acc[0], acc[1], W1p, b1.reshape(1, D), W2, b2.reshape(1, D),
      W2p, b2p.reshape(1, D))


def _tc_mlp_last(xp, acc, W1p, b1, W2, b2):
    row_spec = pl.BlockSpec((R, D), lambda i: (i, 0))
    full_spec = pl.BlockSpec((D, D), lambda i: (0, 0))
    bias_spec = pl.BlockSpec((1, D), lambda i: (0, 0))
    return pl.pallas_call(
        _mlp_last_body,
        grid=(GRID,),
        in_specs=[row_spec, row_spec, row_spec, full_spec, bias_spec, full_spec, bias_spec],
        out_specs=row_spec,
        out_shape=jax.ShapeDtypeStruct((NP, D), jnp.float32),
    )(xp, acc[0], acc[1], W1p, b1.reshape(1, D), W2, b2.reshape(1, D))


def _pack_table(h_bf16):
    return lax.bitcast_convert_type(h_bf16.reshape(NP, HW, 2), jnp.int32)


def kernel(x, edge_index, W1_0, b1_0, W2_0, b2_0, W1_1, b1_1, W2_1, b2_1,
           W1_2, b1_2, W2_2, b2_2):
    perm = jnp.asarray(_PERM)
    src = edge_index[0]
    dst = edge_index[1]
    src_p = jnp.full((EP,), N, dtype=jnp.int32).at[:E].set(src)
    dst_p = jnp.zeros((EP,), dtype=jnp.int32).at[:E].set(dst)
    src_g = src_p.reshape(NW, K, B)
    dst_g = dst_p.reshape(NW, K, B)
    zeros = jnp.zeros((NP, D), dtype=jnp.float32)

    h = jnp.zeros((NP, D), dtype=jnp.float32).at[:N].set(x)
    table = _pack_table(h.astype(jnp.bfloat16))
    xp = jnp.take(h, perm, axis=1)

    weights = [(W1_0, b1_0, W2_0, b2_0), (W1_1, b1_1, W2_1, b2_1), (W1_2, b1_2, W2_2, b2_2)]
    out = None
    for l, (W1, b1, W2, b2) in enumerate(weights):
        acc = _sc_aggregate(table, src_g, dst_g, zeros)
        W1p = jnp.take(W1, perm, axis=0)
        if l < 2:
            W2p = jnp.take(W2, perm, axis=1)
            b2p = jnp.take(b2, perm, axis=0)
            h_bf, xp = _tc_mlp_mid(xp, acc, W1p, b1, W2, b2, W2p, b2p)
            table = _pack_table(h_bf)
        else:
            out = _tc_mlp_last(xp, acc, W1p, b1, W2, b2)
    return out[:N]
